# baseline (speedup 1.0000x reference)
import functools

import jax
import jax.numpy as jnp
from jax import lax
from jax.experimental import pallas as pl
from jax.experimental.pallas import tpu as pltpu
from jax.experimental.pallas import tpu_sc as plsc

N, P, C = 32, 8732, 81
PPAD = 8736
NV = PPAD // 16
L4 = P * 4
L4PAD = NV * 64

_CH = 2048


def _tc_body(conf_ref, lab_ref, ploc_ref, gloc_ref,
             negp_ref, nll_ref, stats_ref):
    num_pos = jnp.float32(0.0)
    pos_nll = jnp.float32(0.0)
    pos_hub = jnp.float32(0.0)
    eye_c = jnp.eye(C, dtype=jnp.float32)
    eye_4 = jnp.eye(4, dtype=jnp.float32)
    dn = (((1,), (1,)), ((), ()))
    for j in range(0, P, _CH):
        h = min(_CH, P - j)
        c_raw = conf_ref[0, pl.ds(j, h), :]
        ct = lax.dot_general(eye_c, c_raw, dn,
                             preferred_element_type=jnp.float32)
        lab = lab_ref[0, :, pl.ds(j, h)]
        et = jnp.exp(ct)
        srow = jnp.sum(et, axis=0, keepdims=True)
        sub_iota = lax.broadcasted_iota(jnp.int32, (C, h), 0)
        msel = jnp.where(sub_iota == lab, ct, 0.0)
        clrow = jnp.sum(msel, axis=0, keepdims=True)
        lse = jnp.log(srow)
        nll = lse - clrow
        pos = lab > 0
        negp = jnp.where(pos, -1.0, lse - ct[0:1, :])
        negp_ref[0, :, pl.ds(j, h)] = negp
        nll_ref[0, :, pl.ds(j, h)] = nll

        pt = lax.dot_general(eye_4, ploc_ref[0, pl.ds(j, h), :], dn,
                             preferred_element_type=jnp.float32)
        gt = lax.dot_general(eye_4, gloc_ref[0, pl.ds(j, h), :], dn,
                             preferred_element_type=jnp.float32)
        d = pt - gt
        ad = jnp.abs(d)
        hub = jnp.where(ad < 1.0, 0.5 * d * d, ad - 0.5)
        hrow = jnp.sum(hub, axis=0, keepdims=True)
        posf = jnp.where(pos, 1.0, 0.0)
        num_pos += jnp.sum(posf)
        pos_nll += jnp.sum(nll * posf)
        pos_hub += jnp.sum(hrow * posf)

    negp_ref[0, :, pl.ds(P, PPAD - P)] = jnp.full((1, PPAD - P), -1.0, jnp.float32)
    nll_ref[0, :, pl.ds(P, PPAD - P)] = jnp.zeros((1, PPAD - P), jnp.float32)
    lane8 = lax.broadcasted_iota(jnp.int32, (1, 8), 1)
    stats_ref[0] = jnp.where(lane8 == 0, num_pos,
                   jnp.where(lane8 == 1, pos_nll,
                   jnp.where(lane8 == 2, pos_hub, 0.0)))


def _tc_stage(conf_t, lab3, ploc_t, gloc_t):
    return pl.pallas_call(
        _tc_body,
        grid=(N,),
        in_specs=[
            pl.BlockSpec((1, P, C), lambda i: (i, 0, 0)),
            pl.BlockSpec((1, 1, P), lambda i: (i, 0, 0)),
            pl.BlockSpec((1, P, 4), lambda i: (i, 0, 0)),
            pl.BlockSpec((1, P, 4), lambda i: (i, 0, 0)),
        ],
        out_specs=[
            pl.BlockSpec((1, 1, PPAD), lambda i: (i, 0, 0)),
            pl.BlockSpec((1, 1, PPAD), lambda i: (i, 0, 0)),
            pl.BlockSpec((1, 1, 8), lambda i: (i, 0, 0)),
        ],
        out_shape=[
            jax.ShapeDtypeStruct((N, 1, PPAD), jnp.float32),
            jax.ShapeDtypeStruct((N, 1, PPAD), jnp.float32),
            jax.ShapeDtypeStruct((N, 1, 8), jnp.float32),
        ],
    )(conf_t, lab3, ploc_t, gloc_t)


def _sc_mine_body(negp_hbm, nll_hbm, npos_hbm, out_hbm,
                  negp_v, nll_v, npos_v, out_v):
    wid = lax.axis_index("s") * 2 + lax.axis_index("c")
    iota = lax.broadcasted_iota(jnp.int32, (16,), 0)

    def vtot(x):
        for k in (1, 2, 4, 8):
            x = x + x[iota ^ k]
        return x[0]

    pltpu.sync_copy(negp_hbm.at[pl.ds(wid * PPAD, PPAD)], negp_v)
    pltpu.sync_copy(nll_hbm.at[pl.ds(wid * PPAD, PPAD)], nll_v)
    base16 = jnp.where(wid >= 16, 16, 0)
    pltpu.sync_copy(npos_hbm.at[pl.ds(base16, 16)], npos_v)
    lane = wid - base16
    npos = vtot(jnp.where(iota == lane, npos_v[...], 0.0))

    def pass1(i, carry):
        cnt, sneg = carry
        x = negp_v[pl.ds(i * 16, 16)]
        nl = nll_v[pl.ds(i * 16, 16)]
        isneg = x >= 0.0
        return (cnt + jnp.where(isneg, 1.0, 0.0),
                sneg + jnp.where(isneg, nl, 0.0))

    z16 = jnp.zeros((16,), jnp.float32)
    cnt_v, sneg_v = lax.fori_loop(0, NV, pass1, (z16, z16))
    negc = vtot(cnt_v)
    allneg_nll = vtot(sneg_v)
    kneg = jnp.minimum(3.0 * npos, negc)

    def cnt_ge(tf):
        def body(i, acc):
            x = negp_v[pl.ds(i * 16, 16)]
            return acc + jnp.where(x >= tf, 1.0, 0.0)
        return vtot(lax.fori_loop(0, NV, body, z16))

    def slow():
        def bs(i, lohi):
            lo, hi = lohi
            d = hi - lo
            mid = lo + d // 2 + (d & 1)
            midf = lax.bitcast_convert_type(mid, jnp.float32)
            ok = cnt_ge(midf) >= kneg
            return (jnp.where(ok, mid, lo), jnp.where(ok, hi, mid - 1))

        tstar, _ = lax.fori_loop(
            0, 31, bs, (jnp.int32(0), jnp.int32(2**31 - 1)))
        tstarf = lax.bitcast_convert_type(tstar, jnp.float32)
        tnextf = lax.bitcast_convert_type(tstar + 1, jnp.float32)
        r = kneg - cnt_ge(tnextf)

        def cnt_tie(m):
            def body(i, acc):
                x = negp_v[pl.ds(i * 16, 16)]
                tie = (x == tstarf) & (i * 16 + iota <= m)
                return acc + jnp.where(tie, 1.0, 0.0)
            return vtot(lax.fori_loop(0, NV, body, z16))

        def bs2(i, lohi):
            lo, hi = lohi
            mid = (lo + hi) // 2
            ok = cnt_tie(mid) >= r
            return (jnp.where(ok, lo, mid + 1), jnp.where(ok, mid, hi))

        mstar, _ = lax.fori_loop(
            0, 14, bs2, (jnp.int32(0), jnp.int32(PPAD - 1)))

        def sum_sel(i, acc):
            x = negp_v[pl.ds(i * 16, 16)]
            sel = (x > tstarf) | ((x == tstarf) & (i * 16 + iota <= mstar))
            return acc + jnp.where(sel, nll_v[pl.ds(i * 16, 16)], 0.0)

        return vtot(lax.fori_loop(0, NV, sum_sel, z16))

    sel_nll = lax.cond(
        kneg >= negc,
        lambda: allneg_nll,
        lambda: lax.cond(kneg == 0.0, lambda: jnp.float32(0.0), slow))

    row = jnp.where(iota == 0, sel_nll, jnp.where(iota == 1, kneg, 0.0))
    out_v[...] = row
    pltpu.sync_copy(out_v, out_hbm.at[pl.ds(wid * 16, 16)])


@functools.lru_cache(maxsize=1)
def _get_sc_mine():
    return pl.kernel(
        _sc_mine_body,
        out_type=jax.ShapeDtypeStruct((N * 16,), jnp.float32),
        mesh=plsc.VectorSubcoreMesh(core_axis_name="c", subcore_axis_name="s"),
        scratch_types=[
            pltpu.VMEM((PPAD,), jnp.float32),
            pltpu.VMEM((PPAD,), jnp.float32),
            pltpu.VMEM((16,), jnp.float32),
            pltpu.VMEM((16,), jnp.float32),
        ],
    )


def kernel(confidence, pred_loc, gt_class_labels, gt_bbox_loc):
    lab3 = gt_class_labels.astype(jnp.int32).reshape(N, 1, P)
    negp3, nll3, stats3 = _tc_stage(confidence, lab3, pred_loc, gt_bbox_loc)
    stats = stats3.reshape(N, 8)
    num_pos, pos_nll, pos_hub = stats[:, 0], stats[:, 1], stats[:, 2]

    mined = _get_sc_mine()(negp3.reshape(N * PPAD), nll3.reshape(N * PPAD),
                           num_pos).reshape(N, 16)
    sel_nll = jnp.sum(mined[:, 0])
    kneg = jnp.sum(mined[:, 1])
    npos = jnp.sum(num_pos)

    conf_loss = (jnp.sum(pos_nll) + sel_nll) / (npos + kneg)
    loc_loss = jnp.sum(pos_hub) / (npos * 4.0)
    return (conf_loss, loc_loss)
